# baseline (speedup 1.0000x reference)
import jax
import jax.numpy as jnp
from jax import lax
from jax.experimental import pallas as pl
from jax.experimental.pallas import tpu as pltpu
from jax.experimental.pallas import tpu_sc as plsc

_BATCH = 16384
_HIST = 50
_DIM = 64
_CB = 128
_K = _HIST * (_BATCH // _CB)
_NC = 2
_NS = 16
_NW = _NC * _NS
_K_PER_W = _K // _NW
_TPAD = 129


def _emb_body(idx_hbm, table_hbm, out_hbm, idx_v,
              g0, g1, g2, g3, g4, t0, t1, t2, t3, t4,
              gs0, gs1, gs2, gs3, gs4, ws0, ws1, ws2, ws3, ws4):
    cid = lax.axis_index("c")
    sid = lax.axis_index("s")
    wid = sid * _NC + cid
    kbase = wid * _K_PER_W

    gbufs = (g0, g1, g2, g3, g4)
    tbufs = (t0, t1, t2, t3, t4)
    gsems = (gs0, gs1, gs2, gs3, gs4)
    wsems = (ws0, ws1, ws2, ws3, ws4)

    pltpu.sync_copy(idx_hbm.at[pl.ds(kbase, _K_PER_W)], idx_v)

    iota16 = lax.iota(jnp.int32, 16)

    qrows = [iota16 + 16 * q for q in range(4)]

    def ig(kk, b):
        pltpu.async_copy(table_hbm.at[idx_v.at[kk]], gbufs[b], gsems[b])

    def wg(b):
        pltpu.make_async_copy(
            table_hbm.at[pl.ds(0, _CB)], gbufs[b], gsems[b]
        ).wait()

    def transpose(b):
        gb, tb = gbufs[b], tbufs[b]

        @pl.loop(0, _CB, unroll=4)
        def _(j):
            col = iota16 * 0 + j
            for q in range(4):
                v = gb[j, pl.ds(16 * q, 16)]
                plsc.store_scatter(tb, [qrows[q], col], v)

    def iw(kk, b):
        k = kbase + kk
        h = k // _CB
        c = lax.rem(k, _CB)
        for q in range(8):
            pltpu.async_copy(
                tbufs[b].at[pl.ds(q * 8, 8), pl.ds(0, _CB)],
                out_hbm.at[h, q, c],
                wsems[b],
            )

    def ww(b):
        for q in range(8):
            pltpu.make_async_copy(
                tbufs[b].at[pl.ds(q * 8, 8), pl.ds(0, _CB)],
                out_hbm.at[0, q, 0],
                wsems[b],
            ).wait()

    def slot(kk, b, do_ww=True, do_ig=True):
        wg(b)
        if do_ww:
            ww(b)
        transpose(b)
        if do_ig:
            ig(kk + 5, b)
        iw(kk, b)

    for b in range(5):
        ig(b, b)
    for kk in range(5):
        slot(kk, kk, do_ww=False)

    @pl.loop(5, _K_PER_W - 5, step=5)
    def _(k0):
        for b in range(5):
            slot(k0 + b, b)

    for b in range(5):
        slot(_K_PER_W - 5 + b, b, do_ig=False)
    for b in range(5):
        ww(b)


@jax.jit
def kernel(token_ids, weights):
    idx = token_ids.T.astype(jnp.int32).reshape(_K, _CB)
    run = pl.kernel(
        _emb_body,
        out_type=jax.ShapeDtypeStruct((_HIST, 8, _BATCH // _CB, 8, _CB),
                                      jnp.float32),
        mesh=plsc.VectorSubcoreMesh(core_axis_name="c", subcore_axis_name="s"),
        scratch_types=[
            pltpu.VMEM((_K_PER_W, _CB), jnp.int32),
            pltpu.VMEM((_CB, _DIM), jnp.float32),
            pltpu.VMEM((_CB, _DIM), jnp.float32),
            pltpu.VMEM((_CB, _DIM), jnp.float32),
            pltpu.VMEM((_CB, _DIM), jnp.float32),
            pltpu.VMEM((_CB, _DIM), jnp.float32),
            pltpu.VMEM((_DIM, _TPAD), jnp.float32),
            pltpu.VMEM((_DIM, _TPAD), jnp.float32),
            pltpu.VMEM((_DIM, _TPAD), jnp.float32),
            pltpu.VMEM((_DIM, _TPAD), jnp.float32),
            pltpu.VMEM((_DIM, _TPAD), jnp.float32),
            pltpu.SemaphoreType.DMA,
            pltpu.SemaphoreType.DMA,
            pltpu.SemaphoreType.DMA,
            pltpu.SemaphoreType.DMA,
            pltpu.SemaphoreType.DMA,
            pltpu.SemaphoreType.DMA,
            pltpu.SemaphoreType.DMA,
            pltpu.SemaphoreType.DMA,
            pltpu.SemaphoreType.DMA,
            pltpu.SemaphoreType.DMA,
        ],
        compiler_params=pltpu.CompilerParams(
            use_tc_tiling_on_sc=False, needs_layout_passes=False
        ),
    )
    out5 = run(idx, weights)
    return out5.transpose((2, 4, 0, 1, 3)).reshape(_BATCH, _HIST, _DIM)


# speedup vs baseline: 1.0042x; 1.0042x over previous
import jax
import jax.numpy as jnp
from jax import lax
from jax.experimental import pallas as pl
from jax.experimental.pallas import tpu as pltpu
from jax.experimental.pallas import tpu_sc as plsc

_BATCH = 16384
_HIST = 50
_DIM = 64
_CB = 128
_K = _HIST * (_BATCH // _CB)
_NC = 2
_NS = 16
_NW = _NC * _NS
_K_PER_W = _K // _NW
_TPAD = 129


def _emb_body(idx_hbm, table_hbm, out_hbm, idx_v,
              g0, g1, g2, t0, t1, t2, gs0, gs1, gs2, ws0, ws1, ws2):
    cid = lax.axis_index("c")
    sid = lax.axis_index("s")
    wid = sid * _NC + cid
    kbase = wid * _K_PER_W

    gbufs = (g0, g1, g2)
    tbufs = (t0, t1, t2)
    gsems = (gs0, gs1, gs2)
    wsems = (ws0, ws1, ws2)

    pltpu.sync_copy(idx_hbm.at[pl.ds(kbase, _K_PER_W)], idx_v)

    iota16 = lax.iota(jnp.int32, 16)

    qrows = [iota16 + 16 * q for q in range(4)]

    def ig(kk, b):
        pltpu.async_copy(table_hbm.at[idx_v.at[kk]], gbufs[b], gsems[b])

    def wg(b):
        pltpu.make_async_copy(
            table_hbm.at[pl.ds(0, _CB)], gbufs[b], gsems[b]
        ).wait()

    def transpose(b):
        gb, tb = gbufs[b], tbufs[b]

        @pl.loop(0, _CB, unroll=4)
        def _(j):
            col = iota16 * 0 + j
            for q in range(4):
                v = gb[j, pl.ds(16 * q, 16)]
                plsc.store_scatter(tb, [qrows[q], col], v)

    def iw(kk, b):
        k = kbase + kk
        h = k // _CB
        c = lax.rem(k, _CB)
        for q in range(8):
            pltpu.async_copy(
                tbufs[b].at[pl.ds(q * 8, 8), pl.ds(0, _CB)],
                out_hbm.at[h, q, c],
                wsems[b],
            )

    def ww(b):
        for q in range(8):
            pltpu.make_async_copy(
                tbufs[b].at[pl.ds(q * 8, 8), pl.ds(0, _CB)],
                out_hbm.at[0, q, 0],
                wsems[b],
            ).wait()

    def slot(kk, b, do_ww=True, do_ig=True):
        wg(b)
        if do_ww:
            ww((b + 1) % 3)
        transpose(b)
        if do_ig:
            ig(kk + 3, b)
        iw(kk, b)

    ig(0, 0)
    ig(1, 1)
    ig(2, 2)
    slot(0, 0, do_ww=False)
    slot(1, 1, do_ww=False)

    @pl.loop(2, _K_PER_W - 3, step=3)
    def _(k0):
        slot(k0, 2)
        slot(k0 + 1, 0)
        slot(k0 + 2, 1)

    slot(_K_PER_W - 3, 2, do_ig=False)
    slot(_K_PER_W - 2, 0, do_ig=False)
    slot(_K_PER_W - 1, 1, do_ig=False)
    ww(0)
    ww(1)


@jax.jit
def kernel(token_ids, weights):
    idx = token_ids.T.astype(jnp.int32).reshape(_K, _CB)
    run = pl.kernel(
        _emb_body,
        out_type=jax.ShapeDtypeStruct((_HIST, 8, _BATCH // _CB, 8, _CB),
                                      jnp.float32),
        mesh=plsc.VectorSubcoreMesh(core_axis_name="c", subcore_axis_name="s"),
        scratch_types=[
            pltpu.VMEM((_K_PER_W, _CB), jnp.int32),
            pltpu.VMEM((_CB, _DIM), jnp.float32),
            pltpu.VMEM((_CB, _DIM), jnp.float32),
            pltpu.VMEM((_CB, _DIM), jnp.float32),
            pltpu.VMEM((_DIM, _TPAD), jnp.float32),
            pltpu.VMEM((_DIM, _TPAD), jnp.float32),
            pltpu.VMEM((_DIM, _TPAD), jnp.float32),
            pltpu.SemaphoreType.DMA,
            pltpu.SemaphoreType.DMA,
            pltpu.SemaphoreType.DMA,
            pltpu.SemaphoreType.DMA,
            pltpu.SemaphoreType.DMA,
            pltpu.SemaphoreType.DMA,
        ],
        compiler_params=pltpu.CompilerParams(
            use_tc_tiling_on_sc=False, needs_layout_passes=False
        ),
    )
    out5 = run(idx, weights)
    return out5.transpose((2, 4, 0, 1, 3)).reshape(_BATCH, _HIST, _DIM)
